# Initial kernel scaffold; baseline (speedup 1.0000x reference)
import functools

import jax
import jax.numpy as jnp
from jax import lax
from jax.experimental import pallas as pl
from jax.experimental.pallas import tpu as pltpu
from jax.experimental.pallas import tpu_sc as plsc

N = 10000
E = 320000
D = 128
D_OUT = 64
EPS = 1e-5

NC, NS, L = 2, 16, 16
NW = NC * NS
C = 128
EPT = ((E // NW) + C - 1) // C * C
NCH = EPT // C
EPAD = EPT * NW
EPT_DEG = EPAD // NS
NCH_DEG = EPT_DEG // C
NPAD = 10240
NSL = NPAD // NS
NROWS = N // NS

_MESH = plsc.VectorSubcoreMesh(core_axis_name="c", subcore_axis_name="s")


def _prep_body(row_hbm, col_hbm, w_hbm, dis_hbm, s_hbm,
               deg_sh, dis_sh, colb, wb, rowb, degb, disb, disloc, sb):
    cid = lax.axis_index("c")
    sid = lax.axis_index("s")
    wid = sid * NC + cid

    def zstep(i, _):
        degb[pl.ds(i * L, L)] = jnp.zeros((L,), jnp.float32)
        return 0
    lax.fori_loop(0, NSL // L, zstep, 0)
    pltpu.sync_copy(degb, deg_sh.at[pl.ds(sid * NSL, NSL)])
    plsc.subcore_barrier()

    def dstep(k, _):
        base = sid * EPT_DEG + k * C
        pltpu.sync_copy(col_hbm.at[pl.ds(base, C)], colb)
        pltpu.sync_copy(w_hbm.at[pl.ds(base, C)], wb)
        pltpu.sync_copy(wb, deg_sh.at[colb], add=True)
        return 0
    lax.fori_loop(0, NCH_DEG, dstep, 0)
    plsc.subcore_barrier()

    pltpu.sync_copy(deg_sh.at[pl.ds(sid * NSL, NSL)], degb)

    def nstep(i, _):
        d = degb[pl.ds(i * L, L)]
        xi = plsc.bitcast(d, jnp.int32)
        yi = jnp.int32(0x5F3759DF) - (xi >> 1)
        y = plsc.bitcast(yi, jnp.float32)
        y = y * (1.5 - 0.5 * d * y * y)
        y = y * (1.5 - 0.5 * d * y * y)
        y = y * (1.5 - 0.5 * d * y * y)
        y = y * (1.5 - 0.5 * d * y * y)
        y = jnp.where(d > 0.0, y, 0.0)
        disb[pl.ds(i * L, L)] = y
        return 0
    lax.fori_loop(0, NSL // L, nstep, 0)
    pltpu.sync_copy(disb, dis_sh.at[pl.ds(sid * NSL, NSL)])

    @pl.when(cid == 0)
    def _():
        pltpu.sync_copy(disb, dis_hbm.at[pl.ds(sid * NSL, NSL)])

    plsc.subcore_barrier()
    pltpu.sync_copy(dis_sh, disloc)

    def sstep(k, _):
        base = wid * EPT + k * C
        pltpu.sync_copy(row_hbm.at[pl.ds(base, C)], rowb)
        pltpu.sync_copy(w_hbm.at[pl.ds(base, C)], wb)
        for j in range(C // L):
            idx = rowb[pl.ds(j * L, L)]
            dv = plsc.load_gather(disloc, [idx])
            sb[pl.ds(j * L, L)] = wb[pl.ds(j * L, L)] * dv
        pltpu.sync_copy(sb, s_hbm.at[pl.ds(base, C)])
        return 0
    lax.fori_loop(0, NCH, sstep, 0)


_prep = pl.kernel(
    _prep_body,
    out_type=(jax.ShapeDtypeStruct((NPAD,), jnp.float32),
              jax.ShapeDtypeStruct((EPAD,), jnp.float32)),
    mesh=_MESH,
    scratch_types=[
        pltpu.VMEM_SHARED((NPAD,), jnp.float32),
        pltpu.VMEM_SHARED((NPAD,), jnp.float32),
        pltpu.VMEM((C,), jnp.int32),
        pltpu.VMEM((C,), jnp.float32),
        pltpu.VMEM((C,), jnp.int32),
        pltpu.VMEM((NSL,), jnp.float32),
        pltpu.VMEM((NSL,), jnp.float32),
        pltpu.VMEM((NPAD,), jnp.float32),
        pltpu.VMEM((C,), jnp.float32),
    ],
)


def _conv_body(h_hbm, row_hbm, col_hbm, s_hbm, part_hbm,
               acc_sh, rowb, colb, sb, buf, sem):
    cid = lax.axis_index("c")
    sid = lax.axis_index("s")
    wid = sid * NC + cid

    def zstep(r, _):
        for j in range(D // L):
            buf[r, pl.ds(j * L, L)] = jnp.zeros((L,), jnp.float32)
        return 0
    lax.fori_loop(0, C, zstep, 0)
    for i in range(5):
        pltpu.sync_copy(buf.at[pl.ds(0, 125)],
                        acc_sh.at[pl.ds(sid * NROWS + i * 125, 125)])
    plsc.subcore_barrier()

    def step(k, _):
        base = wid * EPT + k * C
        pltpu.sync_copy(row_hbm.at[pl.ds(base, C)], rowb)
        pltpu.sync_copy(col_hbm.at[pl.ds(base, C)], colb)
        pltpu.sync_copy(s_hbm.at[pl.ds(base, C)], sb)
        pltpu.async_copy(h_hbm.at[rowb], buf, sem).wait()

        def scale(r, _):
            sv = sb[r]
            for j in range(D // L):
                buf[r, pl.ds(j * L, L)] = buf[r, pl.ds(j * L, L)] * sv
            return 0
        lax.fori_loop(0, C, scale, 0)
        pltpu.sync_copy(buf, acc_sh.at[colb], add=True)
        return 0
    lax.fori_loop(0, NCH, step, 0)
    plsc.subcore_barrier()
    pltpu.sync_copy(acc_sh.at[pl.ds(sid * NROWS, NROWS)],
                    part_hbm.at[cid, pl.ds(sid * NROWS, NROWS)])


_conv = pl.kernel(
    _conv_body,
    out_type=jax.ShapeDtypeStruct((NC, N, D), jnp.float32),
    mesh=_MESH,
    scratch_types=[
        pltpu.VMEM_SHARED((N, D), jnp.float32),
        pltpu.VMEM((C,), jnp.int32),
        pltpu.VMEM((C,), jnp.int32),
        pltpu.VMEM((C,), jnp.float32),
        pltpu.VMEM((C, D), jnp.float32),
        pltpu.SemaphoreType.DMA,
    ],
)


def _mm_body(x_ref, w_ref, o_ref):
    o_ref[...] = jnp.dot(x_ref[...], w_ref[...],
                         preferred_element_type=jnp.float32,
                         precision=lax.Precision.HIGHEST)


def _mid_body(p_ref, dis_ref, b1_ref, g_ref, be_ref, a_ref, w2_ref, o_ref):
    h = dis_ref[...] * (p_ref[0] + p_ref[1]) + b1_ref[...]
    m = jnp.mean(h, axis=0, keepdims=True)
    cen = h - m
    v = jnp.mean(cen * cen, axis=0, keepdims=True)
    hn = cen * lax.rsqrt(v + EPS) * g_ref[...] + be_ref[...]
    a = a_ref[0, 0]
    hp = jnp.where(hn >= 0.0, hn, a * hn)
    o_ref[...] = jnp.dot(hp, w2_ref[...],
                         preferred_element_type=jnp.float32,
                         precision=lax.Precision.HIGHEST)


def _fin_body(q_ref, dis_ref, bmu_ref, bls_ref, mu_ref, ls_ref):
    t = dis_ref[...] * (q_ref[0] + q_ref[1])
    mu_ref[...] = t[:, :D_OUT] + bmu_ref[...]
    ls_ref[...] = t[:, D_OUT:] + bls_ref[...]


def kernel(x, edge_index, edge_attr, W1, b1, gamma1, beta1, prelu_a,
           Wmu, bmu, Wls, bls):
    row = edge_index[0].astype(jnp.int32)
    col = edge_index[1].astype(jnp.int32)
    pad = EPAD - E
    row = jnp.concatenate([row, jnp.zeros((pad,), jnp.int32)])
    col = jnp.concatenate([col, jnp.zeros((pad,), jnp.int32)])
    w = jnp.concatenate([edge_attr, jnp.zeros((pad,), jnp.float32)])

    dis_pad, s = _prep(row, col, w)
    dis = dis_pad[:N].reshape(N, 1)

    h1 = pl.pallas_call(
        _mm_body,
        out_shape=jax.ShapeDtypeStruct((N, D), jnp.float32),
    )(x, W1)

    part1 = _conv(h1, row, col, s)

    W2 = jnp.concatenate([Wmu, Wls], axis=1)
    h2 = pl.pallas_call(
        _mid_body,
        out_shape=jax.ShapeDtypeStruct((N, D), jnp.float32),
    )(part1, dis, b1.reshape(1, D), gamma1.reshape(1, D),
      beta1.reshape(1, D), prelu_a.reshape(1, 1), W2)

    part2 = _conv(h2, row, col, s)

    mu, logstd = pl.pallas_call(
        _fin_body,
        out_shape=(jax.ShapeDtypeStruct((N, D_OUT), jnp.float32),
                   jax.ShapeDtypeStruct((N, D_OUT), jnp.float32)),
    )(part2, dis, bmu.reshape(1, D_OUT), bls.reshape(1, D_OUT))

    return (mu, logstd)


# speedup vs baseline: 8.0346x; 8.0346x over previous
import jax
import jax.numpy as jnp
from jax import lax
from jax.experimental import pallas as pl
from jax.experimental.pallas import tpu as pltpu
from jax.experimental.pallas import tpu_sc as plsc

N = 10000
E = 320000
D = 128
D_OUT = 64
EPS = 1e-5

NC, NS, L = 2, 16, 16
NW = NC * NS
C = 128
EPT = ((E // NW) + C - 1) // C * C
NCH = EPT // C
EPAD = EPT * NW
NPAD = 10240
NSL = NPAD // NS
NROWS = N // NS

_MESH = plsc.VectorSubcoreMesh(core_axis_name="c", subcore_axis_name="s")
_SC_PARAMS = pltpu.CompilerParams(needs_layout_passes=False)


def _deg_body(col_hbm, w_hbm, degp_hbm, deg_sh, colb, wb, degb):
    cid = lax.axis_index("c")
    sid = lax.axis_index("s")
    wid = sid * NC + cid

    def zstep(i, _):
        degb[pl.ds(i * L, L)] = jnp.zeros((L,), jnp.float32)
        return 0
    lax.fori_loop(0, NSL // L, zstep, 0)
    pltpu.sync_copy(degb, deg_sh.at[pl.ds(sid * NSL, NSL)])
    plsc.subcore_barrier()

    def dstep(k, _):
        base = wid * EPT + k * C
        pltpu.sync_copy(col_hbm.at[pl.ds(base, C)], colb)
        pltpu.sync_copy(w_hbm.at[pl.ds(base, C)], wb)
        pltpu.sync_copy(wb, deg_sh.at[colb], add=True)
        return 0
    lax.fori_loop(0, NCH, dstep, 0)
    plsc.subcore_barrier()
    pltpu.sync_copy(deg_sh.at[pl.ds(sid * NSL, NSL)],
                    degp_hbm.at[cid, pl.ds(sid * NSL, NSL)])


_deg = pl.kernel(
    _deg_body,
    out_type=jax.ShapeDtypeStruct((NC, NPAD), jnp.float32),
    mesh=_MESH,
    scratch_types=[
        pltpu.VMEM_SHARED((NPAD,), jnp.float32),
        pltpu.VMEM((C,), jnp.int32),
        pltpu.VMEM((C,), jnp.float32),
        pltpu.VMEM((NSL,), jnp.float32),
    ],
    compiler_params=_SC_PARAMS,
)


def _conv_body(h_hbm, row_hbm, col_hbm, w_hbm, dis_hbm, part_hbm,
               acc_sh, rowb, colb, wb, sb, disloc, buf, sem):
    cid = lax.axis_index("c")
    sid = lax.axis_index("s")
    wid = sid * NC + cid

    pltpu.sync_copy(dis_hbm, disloc)

    def zstep(r, _):
        for j in range(D // L):
            buf[r, pl.ds(j * L, L)] = jnp.zeros((L,), jnp.float32)
        return 0
    lax.fori_loop(0, C, zstep, 0)
    for i in range(NSL // C):
        pltpu.sync_copy(buf, acc_sh.at[pl.ds(sid * NSL + i * C, C)])
    plsc.subcore_barrier()

    def step(k, _):
        base = wid * EPT + k * C
        pltpu.sync_copy(row_hbm.at[pl.ds(base, C)], rowb)
        pltpu.sync_copy(col_hbm.at[pl.ds(base, C)], colb)
        pltpu.sync_copy(w_hbm.at[pl.ds(base, C)], wb)
        pltpu.async_copy(h_hbm.at[rowb], buf, sem).wait()

        for j in range(C // L):
            idx = rowb[pl.ds(j * L, L)]
            sb[pl.ds(j * L, L)] = (wb[pl.ds(j * L, L)]
                                   * plsc.load_gather(disloc, [idx]))

        def scale(r, _):
            sv = plsc.load_gather(sb, [jnp.broadcast_to(r, (L,))])
            for j in range(D // L):
                buf[r, pl.ds(j * L, L)] = buf[r, pl.ds(j * L, L)] * sv
            return 0
        lax.fori_loop(0, C, scale, 0)
        pltpu.sync_copy(buf, acc_sh.at[colb], add=True)
        return 0
    lax.fori_loop(0, NCH, step, 0)
    plsc.subcore_barrier()
    pltpu.sync_copy(acc_sh.at[pl.ds(sid * NSL, NSL)],
                    part_hbm.at[cid, pl.ds(sid * NSL, NSL)])


_conv = pl.kernel(
    _conv_body,
    out_type=jax.ShapeDtypeStruct((NC, NPAD, D), jnp.float32),
    mesh=_MESH,
    scratch_types=[
        pltpu.VMEM_SHARED((NPAD, D), jnp.float32),
        pltpu.VMEM((C,), jnp.int32),
        pltpu.VMEM((C,), jnp.int32),
        pltpu.VMEM((C,), jnp.float32),
        pltpu.VMEM((C,), jnp.float32),
        pltpu.VMEM((NPAD,), jnp.float32),
        pltpu.VMEM((C, D), jnp.float32),
        pltpu.SemaphoreType.DMA,
    ],
    compiler_params=_SC_PARAMS,
)


def _mm_body(x_ref, w_ref, o_ref):
    o_ref[...] = jnp.dot(x_ref[...], w_ref[...],
                         preferred_element_type=jnp.float32,
                         precision=lax.Precision.HIGHEST)


def _dis_body(degp_ref, dis_ref):
    deg = degp_ref[0] + degp_ref[1]
    dis_ref[...] = jnp.where(deg > 0.0, lax.rsqrt(jnp.maximum(deg, EPS)), 0.0)


def _mid_body(p_ref, dis_ref, b1_ref, g_ref, be_ref, a_ref, w2_ref, o_ref):
    h = dis_ref[...] * (p_ref[0, :N, :] + p_ref[1, :N, :]) + b1_ref[...]
    m = jnp.mean(h, axis=0, keepdims=True)
    cen = h - m
    v = jnp.mean(cen * cen, axis=0, keepdims=True)
    hn = cen * lax.rsqrt(v + EPS) * g_ref[...] + be_ref[...]
    a = a_ref[0, 0]
    hp = jnp.where(hn >= 0.0, hn, a * hn)
    o_ref[...] = jnp.dot(hp, w2_ref[...],
                         preferred_element_type=jnp.float32,
                         precision=lax.Precision.HIGHEST)


def _fin_body(q_ref, dis_ref, bmu_ref, bls_ref, mu_ref, ls_ref):
    t = dis_ref[...] * (q_ref[0, :N, :] + q_ref[1, :N, :])
    mu_ref[...] = t[:, :D_OUT] + bmu_ref[...]
    ls_ref[...] = t[:, D_OUT:] + bls_ref[...]


def kernel(x, edge_index, edge_attr, W1, b1, gamma1, beta1, prelu_a,
           Wmu, bmu, Wls, bls):
    row = edge_index[0].astype(jnp.int32)
    col = edge_index[1].astype(jnp.int32)
    pad = EPAD - E
    row = jnp.concatenate([row, jnp.zeros((pad,), jnp.int32)])
    col = jnp.concatenate([col, jnp.zeros((pad,), jnp.int32)])
    w = jnp.concatenate([edge_attr, jnp.zeros((pad,), jnp.float32)])

    degp = _deg(col, w)
    h1 = pl.pallas_call(
        _mm_body,
        out_shape=jax.ShapeDtypeStruct((N, D), jnp.float32),
    )(x, W1)

    dis2d = pl.pallas_call(
        _dis_body,
        out_shape=jax.ShapeDtypeStruct((NPAD // D, D), jnp.float32),
    )(degp.reshape(NC, NPAD // D, D))
    dis_flat = dis2d.reshape(NPAD)
    dis = dis_flat[:N].reshape(N, 1)

    part1 = _conv(h1, row, col, w, dis_flat)

    W2 = jnp.concatenate([Wmu, Wls], axis=1)
    h2 = pl.pallas_call(
        _mid_body,
        out_shape=jax.ShapeDtypeStruct((N, D), jnp.float32),
    )(part1, dis, b1.reshape(1, D), gamma1.reshape(1, D),
      beta1.reshape(1, D), prelu_a.reshape(1, 1), W2)

    part2 = _conv(h2, row, col, w, dis_flat)

    mu, logstd = pl.pallas_call(
        _fin_body,
        out_shape=(jax.ShapeDtypeStruct((N, D_OUT), jnp.float32),
                   jax.ShapeDtypeStruct((N, D_OUT), jnp.float32)),
    )(part2, dis, bmu.reshape(1, D_OUT), bls.reshape(1, D_OUT))

    return (mu, logstd)


# speedup vs baseline: 9.0392x; 1.1250x over previous
import jax
import jax.numpy as jnp
from jax import lax
from jax.experimental import pallas as pl
from jax.experimental.pallas import tpu as pltpu
from jax.experimental.pallas import tpu_sc as plsc

N = 10000
E = 320000
D = 128
D_OUT = 64
EPS = 1e-5

NC, NS, L = 2, 16, 16
NW = NC * NS
C = 128
EPT = 10240
NCH = EPT // C
EPAD = EPT * NW
NCHG = EPAD // C
NPAD = 10240
NSL = NPAD // NS
UNROLL = 4

_MESH = plsc.VectorSubcoreMesh(core_axis_name="c", subcore_axis_name="s")
_SC_PARAMS = pltpu.CompilerParams(needs_layout_passes=False)


def _deg_body(col_hbm, w_hbm, degp_hbm, deg_sh, colb, wb, degb):
    cid = lax.axis_index("c")
    sid = lax.axis_index("s")
    wid = sid * NC + cid

    def zstep(i, _):
        degb[pl.ds(i * L, L)] = jnp.zeros((L,), jnp.float32)
        return 0
    lax.fori_loop(0, NSL // L, zstep, 0)
    pltpu.sync_copy(degb, deg_sh.at[pl.ds(sid * NSL, NSL)])
    plsc.subcore_barrier()

    def dstep(k, _):
        base = wid * EPT + k * C
        pltpu.sync_copy(col_hbm.at[pl.ds(base, C)], colb)
        pltpu.sync_copy(w_hbm.at[pl.ds(base, C)], wb)
        pltpu.sync_copy(wb, deg_sh.at[colb], add=True)
        return 0
    lax.fori_loop(0, NCH, dstep, 0)
    plsc.subcore_barrier()
    pltpu.sync_copy(deg_sh.at[pl.ds(sid * NSL, NSL)],
                    degp_hbm.at[cid, pl.ds(sid * NSL, NSL)])


_deg = pl.kernel(
    _deg_body,
    out_type=jax.ShapeDtypeStruct((NC, NPAD), jnp.float32),
    mesh=_MESH,
    scratch_types=[
        pltpu.VMEM_SHARED((NPAD,), jnp.float32),
        pltpu.VMEM((C,), jnp.int32),
        pltpu.VMEM((C,), jnp.float32),
        pltpu.VMEM((NSL,), jnp.float32),
    ],
)


def _conv_body(h_hbm, pk_hbm, dis_hbm, part_hbm,
               acc_sh, idxb, scidx, sb, disloc, buf, gsem, ssem, isem):
    cid = lax.axis_index("c")
    sid = lax.axis_index("s")
    wid = sid * NC + cid
    cbase = wid * NCH

    pltpu.sync_copy(dis_hbm, disloc)

    def zstep(r, _):
        for j in range(D // L):
            buf[0, r, pl.ds(j * L, L)] = jnp.zeros((L,), jnp.float32)
        return 0
    lax.fori_loop(0, C, zstep, 0)
    for i in range(NSL // C):
        pltpu.sync_copy(buf.at[0], acc_sh.at[pl.ds(sid * NSL + i * C, C)])
    plsc.subcore_barrier()

    pltpu.sync_copy(pk_hbm.at[cbase], idxb.at[0])
    pltpu.async_copy(h_hbm.at[idxb.at[0, 0]], buf.at[0], gsem)
    pltpu.async_copy(pk_hbm.at[cbase + 1], idxb.at[1], isem)

    def step(mi, _):
        for p in range(2):
            k = 2 * mi + p
            d, dn = p, 1 - p

            @pl.when(k + 1 < NCH)
            def _():
                pltpu.make_async_copy(pk_hbm.at[cbase], idxb.at[dn],
                                      isem).wait()

                @pl.when(k >= 1)
                def _():
                    pltpu.make_async_copy(buf.at[dn], acc_sh.at[pl.ds(0, C)],
                                          ssem).wait()
                pltpu.async_copy(h_hbm.at[idxb.at[dn, 0]], buf.at[dn], gsem)

            pltpu.make_async_copy(h_hbm.at[idxb.at[d, 0]], buf.at[d],
                                  gsem).wait()

            for j in range(C // L):
                idx = idxb[d, 0, pl.ds(j * L, L)]
                wv = plsc.bitcast(idxb[d, 2, pl.ds(j * L, L)], jnp.float32)
                sb[pl.ds(j * L, L)] = wv * plsc.load_gather(disloc, [idx])
                scidx[d, pl.ds(j * L, L)] = idxb[d, 1, pl.ds(j * L, L)]

            @pl.when(k + 2 < NCH)
            def _():
                pltpu.async_copy(pk_hbm.at[cbase + k + 2], idxb.at[d], isem)

            bufd = buf.at[d]

            def scale(i, _):
                for u in range(UNROLL):
                    r = i * UNROLL + u
                    sv = plsc.load_gather(sb, [jnp.broadcast_to(r, (L,))])
                    for j in range(D // L):
                        bufd[r, pl.ds(j * L, L)] = (bufd[r, pl.ds(j * L, L)]
                                                    * sv)
                return 0
            lax.fori_loop(0, C // UNROLL, scale, 0)

            pltpu.async_copy(bufd, acc_sh.at[scidx.at[d]], ssem, add=True)
        return 0
    lax.fori_loop(0, NCH // 2, step, 0)

    pltpu.make_async_copy(buf.at[0], acc_sh.at[pl.ds(0, C)], ssem).wait()
    pltpu.make_async_copy(buf.at[1], acc_sh.at[pl.ds(0, C)], ssem).wait()
    plsc.subcore_barrier()
    pltpu.sync_copy(acc_sh.at[pl.ds(sid * NSL, NSL)],
                    part_hbm.at[cid, pl.ds(sid * NSL, NSL)])


_conv = pl.kernel(
    _conv_body,
    out_type=jax.ShapeDtypeStruct((NC, NPAD, D), jnp.float32),
    mesh=_MESH,
    scratch_types=[
        pltpu.VMEM_SHARED((NPAD, D), jnp.float32),
        pltpu.VMEM((2, 4, C), jnp.int32),
        pltpu.VMEM((2, C), jnp.int32),
        pltpu.VMEM((C,), jnp.float32),
        pltpu.VMEM((NPAD,), jnp.float32),
        pltpu.VMEM((2, C, D), jnp.float32),
        pltpu.SemaphoreType.DMA,
        pltpu.SemaphoreType.DMA,
        pltpu.SemaphoreType.DMA,
    ],
    compiler_params=_SC_PARAMS,
)


def _mm_body(x_ref, w_ref, o_ref):
    o_ref[...] = jnp.dot(x_ref[...], w_ref[...],
                         preferred_element_type=jnp.float32,
                         precision=lax.Precision.HIGHEST)


def _dis_body(degp_ref, dis_ref):
    deg = degp_ref[0] + degp_ref[1]
    dis_ref[...] = jnp.where(deg > 0.0, lax.rsqrt(jnp.maximum(deg, EPS)), 0.0)


def _mid_body(p_ref, dis_ref, b1_ref, g_ref, be_ref, a_ref, w2_ref, o_ref):
    h = dis_ref[...] * (p_ref[0, :N, :] + p_ref[1, :N, :]) + b1_ref[...]
    m = jnp.mean(h, axis=0, keepdims=True)
    cen = h - m
    v = jnp.mean(cen * cen, axis=0, keepdims=True)
    hn = cen * lax.rsqrt(v + EPS) * g_ref[...] + be_ref[...]
    a = a_ref[0, 0]
    hp = jnp.where(hn >= 0.0, hn, a * hn)
    o_ref[...] = jnp.dot(hp, w2_ref[...],
                         preferred_element_type=jnp.float32,
                         precision=lax.Precision.HIGHEST)


def _fin_body(q_ref, dis_ref, bmu_ref, bls_ref, mu_ref, ls_ref):
    t = dis_ref[...] * (q_ref[0, :N, :] + q_ref[1, :N, :])
    mu_ref[...] = t[:, :D_OUT] + bmu_ref[...]
    ls_ref[...] = t[:, D_OUT:] + bls_ref[...]


def kernel(x, edge_index, edge_attr, W1, b1, gamma1, beta1, prelu_a,
           Wmu, bmu, Wls, bls):
    row = edge_index[0].astype(jnp.int32)
    col = edge_index[1].astype(jnp.int32)
    pad = EPAD - E
    row = jnp.concatenate([row, jnp.zeros((pad,), jnp.int32)])
    col = jnp.concatenate([col, jnp.zeros((pad,), jnp.int32)])
    w = jnp.concatenate([edge_attr, jnp.zeros((pad,), jnp.float32)])
    wbits = lax.bitcast_convert_type(w, jnp.int32)
    packed = jnp.stack([row.reshape(NCHG, C), col.reshape(NCHG, C),
                        wbits.reshape(NCHG, C),
                        jnp.zeros((NCHG, C), jnp.int32)], axis=1)

    degp = _deg(col, w)
    h1 = pl.pallas_call(
        _mm_body,
        out_shape=jax.ShapeDtypeStruct((N, D), jnp.float32),
    )(x, W1)

    dis2d = pl.pallas_call(
        _dis_body,
        out_shape=jax.ShapeDtypeStruct((NPAD // D, D), jnp.float32),
    )(degp.reshape(NC, NPAD // D, D))
    dis_flat = dis2d.reshape(NPAD)
    dis = dis_flat[:N].reshape(N, 1)

    part1 = _conv(h1, packed, dis_flat)

    W2 = jnp.concatenate([Wmu, Wls], axis=1)
    h2 = pl.pallas_call(
        _mid_body,
        out_shape=jax.ShapeDtypeStruct((N, D), jnp.float32),
    )(part1, dis, b1.reshape(1, D), gamma1.reshape(1, D),
      beta1.reshape(1, D), prelu_a.reshape(1, 1), W2)

    part2 = _conv(h2, packed, dis_flat)

    mu, logstd = pl.pallas_call(
        _fin_body,
        out_shape=(jax.ShapeDtypeStruct((N, D_OUT), jnp.float32),
                   jax.ShapeDtypeStruct((N, D_OUT), jnp.float32)),
    )(part2, dis, bmu.reshape(1, D_OUT), bls.reshape(1, D_OUT))

    return (mu, logstd)


# speedup vs baseline: 10.8323x; 1.1984x over previous
import jax
import jax.numpy as jnp
from jax import lax
from jax.experimental import pallas as pl
from jax.experimental.pallas import tpu as pltpu
from jax.experimental.pallas import tpu_sc as plsc

N = 10000
E = 320000
D = 128
D_OUT = 64
EPS = 1e-5

NC, NS, L = 2, 16, 16
NW = NC * NS
C = 128
EPT = 10240
NCH = EPT // C
EPAD = EPT * NW
NCHG = EPAD // C
NPAD = 10240
NSL = NPAD // NS
UNROLL = 4
DEPTH = 4
FH = D // 2
NCH2 = EPAD // C // NS

_MESH = plsc.VectorSubcoreMesh(core_axis_name="c", subcore_axis_name="s")
_SC_PARAMS = pltpu.CompilerParams(needs_layout_passes=False)
_SC_PARAMS_NT = pltpu.CompilerParams(needs_layout_passes=False,
                                     use_tc_tiling_on_sc=False)


def _deg_body(col_hbm, w_hbm, degp_hbm, deg_sh, colb, wb, degb):
    cid = lax.axis_index("c")
    sid = lax.axis_index("s")
    wid = sid * NC + cid

    def zstep(i, _):
        degb[pl.ds(i * L, L)] = jnp.zeros((L,), jnp.float32)
        return 0
    lax.fori_loop(0, NSL // L, zstep, 0)
    pltpu.sync_copy(degb, deg_sh.at[pl.ds(sid * NSL, NSL)])
    plsc.subcore_barrier()

    def dstep(k, _):
        base = wid * EPT + k * C
        pltpu.sync_copy(col_hbm.at[pl.ds(base, C)], colb)
        pltpu.sync_copy(w_hbm.at[pl.ds(base, C)], wb)
        pltpu.sync_copy(wb, deg_sh.at[colb], add=True)
        return 0
    lax.fori_loop(0, NCH, dstep, 0)
    plsc.subcore_barrier()
    pltpu.sync_copy(deg_sh.at[pl.ds(sid * NSL, NSL)],
                    degp_hbm.at[cid, pl.ds(sid * NSL, NSL)])


_deg = pl.kernel(
    _deg_body,
    out_type=jax.ShapeDtypeStruct((NC, NPAD), jnp.float32),
    mesh=_MESH,
    scratch_types=[
        pltpu.VMEM_SHARED((NPAD,), jnp.float32),
        pltpu.VMEM((C,), jnp.int32),
        pltpu.VMEM((C,), jnp.float32),
        pltpu.VMEM((NSL,), jnp.float32),
    ],
)


def _conv_body(h_hbm, pk_hbm, dis_hbm, part_hbm,
               acc_sh, idxb, scidx, sb, disloc, buf, gsem, ssem, isem):
    cid = lax.axis_index("c")
    sid = lax.axis_index("s")
    cbase = sid * NCH2
    hview = h_hbm.at[cid]

    pltpu.sync_copy(dis_hbm, disloc)

    def zstep(r, _):
        for j in range(FH // L):
            buf[0, r, pl.ds(j * L, L)] = jnp.zeros((L,), jnp.float32)
        return 0
    lax.fori_loop(0, C, zstep, 0)
    for i in range(NSL // C):
        pltpu.sync_copy(buf.at[0], acc_sh.at[pl.ds(sid * NSL + i * C, C)])
    plsc.subcore_barrier()

    pltpu.sync_copy(pk_hbm.at[cbase], idxb.at[0])
    pltpu.async_copy(hview.at[idxb.at[0, 0]], buf.at[0], gsem)
    pltpu.async_copy(pk_hbm.at[cbase + 1], idxb.at[1], isem)

    def step(mi, _):
        for p in range(DEPTH):
            k = DEPTH * mi + p
            d = p
            dn = (p + 1) % DEPTH
            dp = (p + 2) % DEPTH

            @pl.when(k + 1 < NCH2)
            def _():
                pltpu.make_async_copy(pk_hbm.at[cbase], idxb.at[dn],
                                      isem).wait()

                @pl.when(k >= DEPTH - 1)
                def _():
                    pltpu.make_async_copy(buf.at[dn], acc_sh.at[pl.ds(0, C)],
                                          ssem).wait()
                pltpu.async_copy(hview.at[idxb.at[dn, 0]], buf.at[dn], gsem)

            pltpu.make_async_copy(hview.at[idxb.at[d, 0]], buf.at[d],
                                  gsem).wait()

            for j in range(C // L):
                idx = idxb[d, 0, pl.ds(j * L, L)]
                wv = plsc.bitcast(idxb[d, 2, pl.ds(j * L, L)], jnp.float32)
                sb[pl.ds(j * L, L)] = wv * plsc.load_gather(disloc, [idx])
                scidx[d, pl.ds(j * L, L)] = idxb[d, 1, pl.ds(j * L, L)]

            @pl.when(k + 2 < NCH2)
            def _():
                pltpu.async_copy(pk_hbm.at[cbase + k + 2], idxb.at[dp], isem)

            bufd = buf.at[d]

            def scale(i, _):
                for u in range(UNROLL):
                    r = i * UNROLL + u
                    sv = plsc.load_gather(sb, [jnp.broadcast_to(r, (L,))])
                    for j in range(FH // L):
                        bufd[r, pl.ds(j * L, L)] = (bufd[r, pl.ds(j * L, L)]
                                                    * sv)
                return 0
            lax.fori_loop(0, C // UNROLL, scale, 0)

            pltpu.async_copy(bufd, acc_sh.at[scidx.at[d]], ssem, add=True)
        return 0
    lax.fori_loop(0, NCH2 // DEPTH, step, 0)

    for d in range(DEPTH):
        pltpu.make_async_copy(buf.at[d], acc_sh.at[pl.ds(0, C)], ssem).wait()
    plsc.subcore_barrier()
    pltpu.sync_copy(acc_sh.at[pl.ds(sid * NSL, NSL)],
                    part_hbm.at[cid, pl.ds(sid * NSL, NSL)])


_conv = pl.kernel(
    _conv_body,
    out_type=jax.ShapeDtypeStruct((NC, NPAD, FH), jnp.float32),
    mesh=_MESH,
    scratch_types=[
        pltpu.VMEM_SHARED((NPAD, FH), jnp.float32),
        pltpu.VMEM((DEPTH, 4, C), jnp.int32),
        pltpu.VMEM((DEPTH, C), jnp.int32),
        pltpu.VMEM((C,), jnp.float32),
        pltpu.VMEM((NPAD,), jnp.float32),
        pltpu.VMEM((DEPTH, C, FH), jnp.float32),
        pltpu.SemaphoreType.DMA,
        pltpu.SemaphoreType.DMA,
        pltpu.SemaphoreType.DMA,
    ],
    compiler_params=_SC_PARAMS_NT,
)


def _mm_body(x_ref, w_ref, o_ref):
    r = jnp.dot(x_ref[...], w_ref[...],
                preferred_element_type=jnp.float32,
                precision=lax.Precision.HIGHEST)
    o_ref[0, :, :] = r[:, :FH]
    o_ref[1, :, :] = r[:, FH:]


def _dis_body(degp_ref, dis_ref):
    deg = degp_ref[0] + degp_ref[1]
    dis_ref[...] = jnp.where(deg > 0.0, lax.rsqrt(jnp.maximum(deg, EPS)), 0.0)


def _mid_body(p_ref, dis_ref, b1_ref, g_ref, be_ref, a_ref, w2_ref, o_ref):
    hcat = jnp.concatenate([p_ref[0, :N, :], p_ref[1, :N, :]], axis=1)
    h = dis_ref[...] * hcat + b1_ref[...]
    m = jnp.mean(h, axis=0, keepdims=True)
    cen = h - m
    v = jnp.mean(cen * cen, axis=0, keepdims=True)
    hn = cen * lax.rsqrt(v + EPS) * g_ref[...] + be_ref[...]
    a = a_ref[0, 0]
    hp = jnp.where(hn >= 0.0, hn, a * hn)
    r = jnp.dot(hp, w2_ref[...],
                preferred_element_type=jnp.float32,
                precision=lax.Precision.HIGHEST)
    o_ref[0, :, :] = r[:, :FH]
    o_ref[1, :, :] = r[:, FH:]


def _fin_body(q_ref, dis_ref, bmu_ref, bls_ref, mu_ref, ls_ref):
    mu_ref[...] = dis_ref[...] * q_ref[0, :N, :] + bmu_ref[...]
    ls_ref[...] = dis_ref[...] * q_ref[1, :N, :] + bls_ref[...]


def kernel(x, edge_index, edge_attr, W1, b1, gamma1, beta1, prelu_a,
           Wmu, bmu, Wls, bls):
    row = edge_index[0].astype(jnp.int32)
    col = edge_index[1].astype(jnp.int32)
    pad = EPAD - E
    row = jnp.concatenate([row, jnp.zeros((pad,), jnp.int32)])
    col = jnp.concatenate([col, jnp.zeros((pad,), jnp.int32)])
    w = jnp.concatenate([edge_attr, jnp.zeros((pad,), jnp.float32)])
    wbits = lax.bitcast_convert_type(w, jnp.int32)
    packed = jnp.stack([row.reshape(NCHG, C), col.reshape(NCHG, C),
                        wbits.reshape(NCHG, C),
                        jnp.zeros((NCHG, C), jnp.int32)], axis=1)

    degp = _deg(col, w)
    h1 = pl.pallas_call(
        _mm_body,
        out_shape=jax.ShapeDtypeStruct((NC, N, FH), jnp.float32),
    )(x, W1)

    dis2d = pl.pallas_call(
        _dis_body,
        out_shape=jax.ShapeDtypeStruct((NPAD // D, D), jnp.float32),
    )(degp.reshape(NC, NPAD // D, D))
    dis_flat = dis2d.reshape(NPAD)
    dis = dis_flat[:N].reshape(N, 1)

    part1 = _conv(h1, packed, dis_flat)

    W2 = jnp.concatenate([Wmu, Wls], axis=1)
    h2 = pl.pallas_call(
        _mid_body,
        out_shape=jax.ShapeDtypeStruct((NC, N, FH), jnp.float32),
    )(part1, dis, b1.reshape(1, D), gamma1.reshape(1, D),
      beta1.reshape(1, D), prelu_a.reshape(1, 1), W2)

    part2 = _conv(h2, packed, dis_flat)

    mu, logstd = pl.pallas_call(
        _fin_body,
        out_shape=(jax.ShapeDtypeStruct((N, D_OUT), jnp.float32),
                   jax.ShapeDtypeStruct((N, D_OUT), jnp.float32)),
    )(part2, dis, bmu.reshape(1, D_OUT), bls.reshape(1, D_OUT))

    return (mu, logstd)
